# baseline (speedup 1.0000x reference)
import functools

import jax
import jax.numpy as jnp
from jax import lax
from jax.experimental import pallas as pl
from jax.experimental.pallas import tpu as pltpu
from jax.experimental.pallas import tpu_sc as plsc

_LANES = 128
_NC, _NS, _L = 2, 16, 16
_NW = _NC * _NS
_M = 512
_CHUNK = 128


def _router_body(tpw, x_ref, rw_ref, rb_ref, eid_ref, tab_ref, xpk_ref):
    xb = x_ref[...]
    half = xb.shape[1] // 2
    r = lax.bitcast_convert_type(xb, jnp.uint32)
    bf = (r + 0x7FFF + ((r >> 16) & 1)) >> 16
    pk = bf[:, :half] | (bf[:, half:] << 16)
    xpk_ref[...] = lax.bitcast_convert_type(pk, jnp.int32)
    logits = jnp.dot(xb, rw_ref[...], preferred_element_type=jnp.float32)
    logits = logits + rb_ref[...]
    m = jnp.max(logits, axis=1, keepdims=True)
    lane = lax.broadcasted_iota(jnp.int32, logits.shape, 1)
    eid = jnp.min(jnp.where(logits >= m, lane, _LANES), axis=1, keepdims=True)
    eid_ref[...] = eid
    lane16 = lax.broadcasted_iota(jnp.int32, (eid.shape[0], _L), 1)
    oh = (eid == lane16).astype(jnp.int32)
    tab_ref[...] = jnp.concatenate(
        [jnp.sum(oh[k * tpw:(k + 1) * tpw], axis=0, keepdims=True)
         for k in range(eid.shape[0] // tpw)], axis=0)[None]


def _dispatch_body(tpw, nchunk, h_off, eid_hbm, table_hbm, x_hbm,
                   xg_hbm, dst_hbm, te_hbm,
                   table_v, eid_v, base_v, dst_v, tile_v,
                   rows_a, rows_b, sin_a, sin_b, sout_a, sout_b):
    wid = lax.axis_index("s") * _NC + lax.axis_index("c")
    base = h_off + wid * tpw
    bufs, sins, souts = [rows_a, rows_b], [sin_a, sin_b], [sout_a, sout_b]
    in_d = {0: pltpu.async_copy(x_hbm.at[pl.ds(base, _CHUNK)], rows_a, sin_a)}
    pltpu.sync_copy(table_hbm, table_v)
    pltpu.sync_copy(eid_hbm.at[pl.ds(base, tpw)], eid_v)

    totals = jnp.zeros((_L,), jnp.int32)
    pref = jnp.zeros((_L,), jnp.int32)
    for w in range(_NW):
        row = table_v[w, :]
        totals = totals + row
        pref = pref + jnp.where(jnp.int32(w) < wid, row, 0)
    ntiles = (totals + (_M - 1)) // _M
    csum = plsc.cumsum(ntiles)
    aoff = (csum - ntiles) * _M
    base_v[...] = aoff + pref

    nl = _CHUNK // _L
    cnt = [jnp.zeros((_L,), jnp.int32) for _ in range(8)]
    for v in range(tpw // _L):
        tok = eid_v[pl.ds(v * _L, _L)]
        rank = jnp.zeros((_L,), jnp.int32)
        for e in range(8):
            msk = tok == e
            cs = plsc.cumsum(jnp.where(msk, 1, 0))
            rank = jnp.where(msk, cnt[e] + cs - 1, rank)
            cnt[e] = cnt[e] + plsc.all_reduce_population_count(msk)
        d = plsc.load_gather(base_v, [tok]) + rank
        dst_v[v // nl, pl.ds((v % nl) * _L, _L)] = d
    pltpu.sync_copy(dst_v, dst_hbm.at[pl.ds(wid * nchunk, nchunk)])

    @pl.when(wid == 0)
    def _():
        lanes = lax.iota(jnp.int32, _L)
        bnd = [jnp.sum(jnp.where(lanes == e, csum, 0)) for e in range(8)]
        for k in range(8):
            iv = lax.iota(jnp.int32, _L) + _L * k
            acc = jnp.zeros((_L,), jnp.int32)
            for e in range(8):
                acc = acc + jnp.where(iv >= bnd[e], 1, 0)
            tile_v[pl.ds(_L * k, _L)] = jnp.minimum(acc, 7)
        pltpu.sync_copy(tile_v, te_hbm)

    out_d = {}
    for j in range(nchunk):
        b = j % 2
        in_d[j].wait()
        if j >= 1:
            out_d[j - 1].wait()
        out_d[j] = pltpu.async_copy(bufs[b], xg_hbm.at[dst_v.at[j]], souts[b])
        if j + 1 < nchunk:
            in_d[j + 1] = pltpu.async_copy(
                x_hbm.at[pl.ds(base + (j + 1) * _CHUNK, _CHUNK)],
                bufs[(j + 1) % 2], sins[(j + 1) % 2])
    out_d[nchunk - 1].wait()


def _expert_mm_body(te_ref, xg_ref, w_ref, yg_ref):
    i = pl.program_id(0)
    pk = lax.bitcast_convert_type(xg_ref[...], jnp.uint32)
    lo = lax.bitcast_convert_type(pk << 16, jnp.float32).astype(jnp.bfloat16)
    hi = lax.bitcast_convert_type(pk & jnp.uint32(0xFFFF0000),
                                  jnp.float32).astype(jnp.bfloat16)
    w = w_ref[te_ref[i]]
    half = w.shape[0] // 2
    yg_ref[...] = (jnp.dot(lo, w[:half], preferred_element_type=jnp.float32)
                   + jnp.dot(hi, w[half:], preferred_element_type=jnp.float32))


def _combine_body(tpw, nchunk, yg1_hbm, yg2_hbm, dst1_hbm, dst2_hbm, y_hbm,
                  dst_v, rows_v, sem):
    wid = lax.axis_index("s") * _NC + lax.axis_index("c")
    hw = _NW // 2

    def do_half(yg_hbm, dst_hbm, lw):
        pltpu.sync_copy(dst_hbm.at[pl.ds(lw * nchunk, nchunk)], dst_v)
        for j in range(nchunk):
            pltpu.async_copy(yg_hbm.at[dst_v.at[j]], rows_v, sem).wait()
            pltpu.sync_copy(rows_v,
                            y_hbm.at[pl.ds(wid * tpw + j * _CHUNK, _CHUNK)])

    @pl.when(wid < hw)
    def _():
        do_half(yg1_hbm, dst1_hbm, wid)

    @pl.when(wid >= hw)
    def _():
        do_half(yg2_hbm, dst2_hbm, wid - hw)


def kernel(x, router_W, router_b, w_c_fc):
    B, T, C = x.shape
    E = w_c_fc.shape[0]
    N = B * T
    N2 = N // 2
    TM = 2048
    TPW = N2 // _NW
    NCHUNK = TPW // _CHUNK
    NT = N2 // _M + E
    NSLOT = NT * _M
    x2 = x.reshape(N, C)

    rw = jnp.zeros((C, _LANES), jnp.float32).at[:, :E].set(router_W)
    rb = jnp.full((1, _LANES), -jnp.inf, jnp.float32).at[0, :E].set(router_b)
    wbf = w_c_fc.astype(jnp.bfloat16)

    eid, table, xpk = pl.pallas_call(
        functools.partial(_router_body, TPW),
        name="router",
        grid=(N // TM,),
        in_specs=[
            pl.BlockSpec((TM, C), lambda i: (i, 0)),
            pl.BlockSpec((C, _LANES), lambda i: (0, 0)),
            pl.BlockSpec((1, _LANES), lambda i: (0, 0)),
        ],
        out_specs=[
            pl.BlockSpec((TM, 1), lambda i: (i, 0)),
            pl.BlockSpec((1, TM // TPW, _L), lambda i: (i, 0, 0)),
            pl.BlockSpec((TM, C // 2), lambda i: (i, 0)),
        ],
        out_shape=(jax.ShapeDtypeStruct((N, 1), jnp.int32),
                   jax.ShapeDtypeStruct((N // TM, TM // TPW, _L), jnp.int32),
                   jax.ShapeDtypeStruct((N, C // 2), jnp.int32)),
    )(x2, rw, rb)
    eid = eid.reshape(N)
    table = table.reshape(N // TPW, _L)

    mesh = plsc.VectorSubcoreMesh(core_axis_name="c", subcore_axis_name="s",
                                  num_cores=_NC, num_subcores=_NS)

    def dispatch(h):
        return pl.kernel(
            functools.partial(_dispatch_body, TPW, NCHUNK, h * N2),
            out_type=(jax.ShapeDtypeStruct((NSLOT, C // 2), jnp.int32),
                      jax.ShapeDtypeStruct((N2 // _CHUNK, _CHUNK), jnp.int32),
                      jax.ShapeDtypeStruct((_LANES,), jnp.int32)),
            mesh=mesh,
            name=f"dispatch{h}",
            compiler_params=pltpu.CompilerParams(needs_layout_passes=False),
            scratch_types=[
                pltpu.VMEM((_NW, _L), jnp.int32),
                pltpu.VMEM((TPW,), jnp.int32),
                pltpu.VMEM((_L,), jnp.int32),
                pltpu.VMEM((NCHUNK, _CHUNK), jnp.int32),
                pltpu.VMEM((_LANES,), jnp.int32),
                pltpu.VMEM((_CHUNK, C // 2), jnp.int32),
                pltpu.VMEM((_CHUNK, C // 2), jnp.int32),
                pltpu.SemaphoreType.DMA,
                pltpu.SemaphoreType.DMA,
                pltpu.SemaphoreType.DMA,
                pltpu.SemaphoreType.DMA,
            ],
        )(eid, lax.slice_in_dim(table, h * _NW, (h + 1) * _NW), xpk)

    def expert_mm(te_h, xg_h):
        return pl.pallas_call(
            _expert_mm_body,
            grid_spec=pltpu.PrefetchScalarGridSpec(
                num_scalar_prefetch=1,
                grid=(NT,),
                in_specs=[
                    pl.BlockSpec((_M, C // 2), lambda i, te_ref: (i, 0)),
                    pl.BlockSpec((E, C, C), lambda i, te_ref: (0, 0, 0)),
                ],
                out_specs=pl.BlockSpec((_M, C), lambda i, te_ref: (i, 0)),
            ),
            out_shape=jax.ShapeDtypeStruct((NSLOT, C), jnp.float32),
        )(te_h, xg_h, wbf)

    xg1, dst1, te1 = dispatch(0)
    xg2, dst2, te2 = dispatch(1)
    yg1 = expert_mm(te1, xg1)
    yg2 = expert_mm(te2, xg2)

    CTPW = N // _NW
    y2 = pl.kernel(
        functools.partial(_combine_body, CTPW, CTPW // _CHUNK),
        out_type=jax.ShapeDtypeStruct((N, C), jnp.float32),
        mesh=mesh,
        name="combine",
        scratch_types=[
            pltpu.VMEM((CTPW // _CHUNK, _CHUNK), jnp.int32),
            pltpu.VMEM((_CHUNK, C), jnp.float32),
            pltpu.SemaphoreType.DMA,
        ],
    )(yg1, yg2, dst1, dst2)

    return y2.reshape(B, T, C)


# speedup vs baseline: 1.0998x; 1.0998x over previous
import functools

import jax
import jax.numpy as jnp
from jax import lax
from jax.experimental import pallas as pl
from jax.experimental.pallas import tpu as pltpu
from jax.experimental.pallas import tpu_sc as plsc

_LANES = 128
_NC, _NS, _L = 2, 16, 16
_NW = _NC * _NS
_M = 512
_CHUNK = 128


def _router_body(tpw, x_ref, rw_ref, rb_ref, eid_ref, tab_ref, xpk_ref):
    xb = x_ref[...]
    half = xb.shape[1] // 2
    r = lax.bitcast_convert_type(xb, jnp.uint32)
    bf = (r + 0x7FFF + ((r >> 16) & 1)) >> 16
    pk = bf[:, :half] | (bf[:, half:] << 16)
    xpk_ref[...] = lax.bitcast_convert_type(pk, jnp.int32)
    logits = jnp.dot(xb, rw_ref[...], preferred_element_type=jnp.float32)
    logits = logits + rb_ref[...]
    m = jnp.max(logits, axis=1, keepdims=True)
    lane = lax.broadcasted_iota(jnp.int32, logits.shape, 1)
    eid = jnp.min(jnp.where(logits >= m, lane, _LANES), axis=1, keepdims=True)
    eid_ref[...] = eid
    lane16 = lax.broadcasted_iota(jnp.int32, (eid.shape[0], _L), 1)
    oh = (eid == lane16).astype(jnp.int32)
    tab_ref[...] = jnp.concatenate(
        [jnp.sum(oh[k * tpw:(k + 1) * tpw], axis=0, keepdims=True)
         for k in range(eid.shape[0] // tpw)], axis=0)[None]


def _dispatch_body(tpw, nchunk, nt, eid_hbm, table_hbm, x_hbm,
                   xg_hbm, dst_hbm, te_hbm, bx_hbm,
                   table_v, eid_v, base_v, dst_v, tile_v, bidx_v,
                   rows_a, rows_b, sin_a, sin_b, sout_a, sout_b):
    wid = lax.axis_index("s") * _NC + lax.axis_index("c")
    base = wid * tpw
    bufs, sins, souts = [rows_a, rows_b], [sin_a, sin_b], [sout_a, sout_b]
    in_d = {0: pltpu.async_copy(x_hbm.at[pl.ds(base, _CHUNK)], rows_a, sin_a)}
    pltpu.sync_copy(table_hbm, table_v)
    pltpu.sync_copy(eid_hbm.at[pl.ds(base, tpw)], eid_v)

    totals = jnp.zeros((_L,), jnp.int32)
    pref = jnp.zeros((_L,), jnp.int32)
    for w in range(_NW):
        row = table_v[w, :]
        totals = totals + row
        pref = pref + jnp.where(jnp.int32(w) < wid, row, 0)
    ntiles = (totals + (_M - 1)) // _M
    csum = plsc.cumsum(ntiles)
    aoff = (csum - ntiles) * _M
    base_v[...] = aoff + pref

    nl = _CHUNK // _L
    cnt = [jnp.zeros((_L,), jnp.int32) for _ in range(8)]
    for v in range(tpw // _L):
        tok = eid_v[pl.ds(v * _L, _L)]
        rank = jnp.zeros((_L,), jnp.int32)
        for e in range(8):
            msk = tok == e
            cs = plsc.cumsum(jnp.where(msk, 1, 0))
            rank = jnp.where(msk, cnt[e] + cs - 1, rank)
            cnt[e] = cnt[e] + plsc.all_reduce_population_count(msk)
        d = plsc.load_gather(base_v, [tok]) + rank
        dst_v[v // nl, pl.ds((v % nl) * _L, _L)] = d
    pltpu.sync_copy(dst_v, dst_hbm.at[pl.ds(wid * nchunk, nchunk)])

    @pl.when(wid == 0)
    def _():
        lanes = lax.iota(jnp.int32, _L)
        bnd = [jnp.sum(jnp.where(lanes == e, csum, 0)) for e in range(8)]
        for k in range(8):
            iv = lax.iota(jnp.int32, _L) + _L * k
            acc = jnp.zeros((_L,), jnp.int32)
            for e in range(8):
                acc = acc + jnp.where(iv >= bnd[e], 1, 0)
            tile_v[pl.ds(_L * k, _L)] = jnp.minimum(acc, 7)
            bidx_v[pl.ds(_L * k, _L)] = jnp.where(iv < bnd[7], iv, nt - 1)
        pltpu.sync_copy(tile_v, te_hbm)
        pltpu.sync_copy(bidx_v, bx_hbm)

    out_d = {}
    for j in range(nchunk):
        b = j % 2
        in_d[j].wait()
        if j >= 1:
            out_d[j - 1].wait()
        out_d[j] = pltpu.async_copy(bufs[b], xg_hbm.at[dst_v.at[j]], souts[b])
        if j + 1 < nchunk:
            in_d[j + 1] = pltpu.async_copy(
                x_hbm.at[pl.ds(base + (j + 1) * _CHUNK, _CHUNK)],
                bufs[(j + 1) % 2], sins[(j + 1) % 2])
    out_d[nchunk - 1].wait()


def _expert_mm_body(te_ref, bx_ref, xg_ref, w_ref, yg_ref):
    i = pl.program_id(0)

    @pl.when(bx_ref[i] == i)
    def _():
        pk = lax.bitcast_convert_type(xg_ref[...], jnp.uint32)
        lo = lax.bitcast_convert_type(pk << 16, jnp.float32).astype(jnp.bfloat16)
        hi = lax.bitcast_convert_type(pk & jnp.uint32(0xFFFF0000),
                                      jnp.float32).astype(jnp.bfloat16)
        w = w_ref[te_ref[i]]
        half = w.shape[0] // 2
        yg_ref[...] = (jnp.dot(lo, w[:half], preferred_element_type=jnp.float32)
                       + jnp.dot(hi, w[half:], preferred_element_type=jnp.float32))


def _combine_body(tpw, nchunk, yg_hbm, dst_hbm, y_hbm, dst_v, rows_v, sem):
    wid = lax.axis_index("s") * _NC + lax.axis_index("c")
    pltpu.sync_copy(dst_hbm.at[pl.ds(wid * nchunk, nchunk)], dst_v)
    for j in range(nchunk):
        pltpu.async_copy(yg_hbm.at[dst_v.at[j]], rows_v, sem).wait()
        pltpu.sync_copy(rows_v, y_hbm.at[pl.ds(wid * tpw + j * _CHUNK, _CHUNK)])


def kernel(x, router_W, router_b, w_c_fc):
    B, T, C = x.shape
    E = w_c_fc.shape[0]
    N = B * T
    TM = 2048
    TPW = N // _NW
    NCHUNK = TPW // _CHUNK
    NT = N // _M + E
    NSLOT = NT * _M
    x2 = x.reshape(N, C)

    rw = jnp.zeros((C, _LANES), jnp.float32).at[:, :E].set(router_W)
    rb = jnp.full((1, _LANES), -jnp.inf, jnp.float32).at[0, :E].set(router_b)
    wbf = w_c_fc.astype(jnp.bfloat16)

    eid, table, xpk = pl.pallas_call(
        functools.partial(_router_body, TPW),
        grid=(N // TM,),
        in_specs=[
            pl.BlockSpec((TM, C), lambda i: (i, 0)),
            pl.BlockSpec((C, _LANES), lambda i: (0, 0)),
            pl.BlockSpec((1, _LANES), lambda i: (0, 0)),
        ],
        out_specs=[
            pl.BlockSpec((TM, 1), lambda i: (i, 0)),
            pl.BlockSpec((1, TM // 512, _L), lambda i: (i, 0, 0)),
            pl.BlockSpec((TM, C // 2), lambda i: (i, 0)),
        ],
        out_shape=(jax.ShapeDtypeStruct((N, 1), jnp.int32),
                   jax.ShapeDtypeStruct((N // TM, TM // 512, _L), jnp.int32),
                   jax.ShapeDtypeStruct((N, C // 2), jnp.int32)),
    )(x2, rw, rb)
    eid = eid.reshape(N)
    table = table.reshape(_NW, _L)

    mesh = plsc.VectorSubcoreMesh(core_axis_name="c", subcore_axis_name="s",
                                  num_cores=_NC, num_subcores=_NS)

    xg, dst, te, bx = pl.kernel(
        functools.partial(_dispatch_body, TPW, NCHUNK, NT),
        out_type=(jax.ShapeDtypeStruct((NSLOT, C // 2), jnp.int32),
                  jax.ShapeDtypeStruct((N // _CHUNK, _CHUNK), jnp.int32),
                  jax.ShapeDtypeStruct((_LANES,), jnp.int32),
                  jax.ShapeDtypeStruct((_LANES,), jnp.int32)),
        mesh=mesh,
        compiler_params=pltpu.CompilerParams(needs_layout_passes=False),
        scratch_types=[
            pltpu.VMEM((_NW, _L), jnp.int32),
            pltpu.VMEM((TPW,), jnp.int32),
            pltpu.VMEM((_L,), jnp.int32),
            pltpu.VMEM((NCHUNK, _CHUNK), jnp.int32),
            pltpu.VMEM((_LANES,), jnp.int32),
            pltpu.VMEM((_LANES,), jnp.int32),
            pltpu.VMEM((_CHUNK, C // 2), jnp.int32),
            pltpu.VMEM((_CHUNK, C // 2), jnp.int32),
            pltpu.SemaphoreType.DMA,
            pltpu.SemaphoreType.DMA,
            pltpu.SemaphoreType.DMA,
            pltpu.SemaphoreType.DMA,
        ],
    )(eid, table, xpk)

    yg = pl.pallas_call(
        _expert_mm_body,
        grid_spec=pltpu.PrefetchScalarGridSpec(
            num_scalar_prefetch=2,
            grid=(NT,),
            in_specs=[
                pl.BlockSpec((_M, C // 2), lambda i, te_ref, bx_ref: (bx_ref[i], 0)),
                pl.BlockSpec((E, C, C), lambda i, te_ref, bx_ref: (0, 0, 0)),
            ],
            out_specs=pl.BlockSpec((_M, C), lambda i, te_ref, bx_ref: (bx_ref[i], 0)),
        ),
        out_shape=jax.ShapeDtypeStruct((NSLOT, C), jnp.float32),
    )(te, bx, xg, wbf)

    y2 = pl.kernel(
        functools.partial(_combine_body, TPW, NCHUNK),
        out_type=jax.ShapeDtypeStruct((N, C), jnp.float32),
        mesh=mesh,
        scratch_types=[
            pltpu.VMEM((NCHUNK, _CHUNK), jnp.int32),
            pltpu.VMEM((_CHUNK, C), jnp.float32),
            pltpu.SemaphoreType.DMA,
        ],
    )(yg, dst)

    return y2.reshape(B, T, C)
